# baseline (speedup 1.0000x reference)
import functools

import jax
import jax.numpy as jnp
from jax import lax
from jax.experimental import pallas as pl
from jax.experimental.pallas import tpu as pltpu
from jax.experimental.pallas import tpu_sc as plsc

N_NODES = 10000
D_FEAT = 128
N_EDGES = 320000

NC = 2
NS = 16
DH = D_FEAT // NC
CH = 128
NCH = 160
E_PAD = NS * NCH * CH
N_PAD = 10240
RPT = N_PAD // NS


def _sc_aggregate(nf_halves, src, dst, w):
    mesh = plsc.VectorSubcoreMesh(core_axis_name="c", subcore_axis_name="s")

    @functools.partial(
        pl.kernel,
        mesh=mesh,
        out_type=jax.ShapeDtypeStruct((NC, N_PAD, DH), jnp.float32),
        compiler_params=pltpu.CompilerParams(use_tc_tiling_on_sc=False),
        scratch_types=[
            pltpu.VMEM_SHARED((N_PAD, DH), jnp.float32),
            pltpu.VMEM_SHARED((N_PAD, DH), jnp.float32),
            pltpu.VMEM((NCH, CH), jnp.int32),
            pltpu.VMEM((CH,), jnp.int32),
            pltpu.VMEM((CH,), jnp.int32),
            pltpu.VMEM((CH,), jnp.float32),
            pltpu.VMEM((CH,), jnp.float32),
            pltpu.VMEM((CH, DH), jnp.float32),
            pltpu.VMEM((CH, DH), jnp.float32),
            pltpu.SemaphoreType.DMA,
            pltpu.SemaphoreType.DMA,
            pltpu.SemaphoreType.DMA,
            pltpu.SemaphoreType.DMA,
        ],
    )
    def body(nf_hbm, src_hbm, dst_hbm, w_hbm, out_hbm, nfs, acc,
             didx, sid0, sid1, wv0, wv1, rows0, rows1,
             semi0, semi1, semg0, semg1):
        c = lax.axis_index("c")
        s = lax.axis_index("s")

        sid = (sid0, sid1)
        wv = (wv0, wv1)
        rows = (rows0, rows1)
        semi = (semi0, semi1)
        semg = (semg0, semg1)

        def start_idx(ci, p):
            off = ci * CH
            pltpu.async_copy(src_hbm.at[s, pl.ds(off, CH)], sid[p], semi[p])
            pltpu.async_copy(w_hbm.at[s, pl.ds(off, CH)], wv[p], semi[p])

        def wait_idx(ci, p):
            off = ci * CH
            pltpu.make_async_copy(
                src_hbm.at[s, pl.ds(off, CH)], sid[p], semi[p]).wait()
            pltpu.make_async_copy(
                w_hbm.at[s, pl.ds(off, CH)], wv[p], semi[p]).wait()

        pltpu.sync_copy(nf_hbm.at[c, pl.ds(s * RPT, RPT)],
                        nfs.at[pl.ds(s * RPT, RPT)])
        pltpu.sync_copy(dst_hbm.at[s], didx)

        zero = jnp.zeros((16,), jnp.float32)

        def zrow(r, carry):
            for k in range(DH // 16):
                rows0[r, pl.ds(k * 16, 16)] = zero
            return carry

        lax.fori_loop(0, CH, zrow, 0)
        for j in range(RPT // CH):
            pltpu.sync_copy(rows0, acc.at[pl.ds(s * RPT + j * CH, CH)])
        plsc.subcore_barrier()

        dnums = lax.GatherDimensionNumbers(
            offset_dims=(), collapsed_slice_dims=(0,),
            start_index_map=(0,))

        def scale(p):
            def grp(g, inner):
                w16 = wv[p][pl.ds(g * 16, 16)]
                for j in range(16):
                    sp = lax.gather(
                        w16, jnp.full((16, 1), j, jnp.int32), dnums,
                        slice_sizes=(1,),
                        mode=lax.GatherScatterMode.PROMISE_IN_BOUNDS)
                    r = g * 16 + j
                    for k in range(DH // 16):
                        rows[p][r, pl.ds(k * 16, 16)] = (
                            rows[p][r, pl.ds(k * 16, 16)] * sp)
                return inner

            lax.fori_loop(0, CH // 16, grp, 0)

        NPAIR = NCH // 2
        start_idx(0, 0)
        start_idx(1, 1)
        wait_idx(0, 0)
        pltpu.async_copy(nfs.at[sid[0]], rows[0], semg[0])

        def step(ci, p, po):
            if p == 0:
                wait_idx(ci + 1, 1)
                pltpu.async_copy(nfs.at[sid[1]], rows[1], semg[1])
            else:
                @pl.when(po != NPAIR - 1)
                def _():
                    wait_idx(ci + 1, 0)
                    pltpu.async_copy(nfs.at[sid[0]], rows[0], semg[0])

            pltpu.make_async_copy(nfs.at[sid[p]], rows[p], semg[p]).wait()
            scale(p)
            pltpu.sync_copy(rows[p], acc.at[didx.at[ci]], add=True)

            @pl.when(po != NPAIR - 1)
            def _():
                start_idx(ci + 2, p)

        def pair(po, carry):
            step(po * 2, 0, po)
            step(po * 2 + 1, 1, po)
            return carry

        lax.fori_loop(0, NPAIR, pair, 0)
        plsc.subcore_barrier()

        pltpu.sync_copy(acc.at[pl.ds(s * RPT, RPT)],
                        out_hbm.at[c, pl.ds(s * RPT, RPT)])

    return body(nf_halves, src, dst, w)


def _tc_body(p_ref, m_ref, b_ref, o_ref):
    agg = jnp.concatenate([p_ref[0], p_ref[1]], axis=1)
    o_ref[...] = jnp.dot(agg, m_ref[...],
                         preferred_element_type=jnp.float32,
                         precision=lax.Precision.HIGHEST) + b_ref[...]


def kernel(n_feat, edge_index, edge_weights, W_in, b_in):
    src = edge_index[0].astype(jnp.int32)
    dst = edge_index[1].astype(jnp.int32)
    w = edge_weights.reshape(-1).astype(jnp.float32)

    pad = E_PAD - N_EDGES
    src = jnp.concatenate([src, jnp.zeros((pad,), jnp.int32)])
    dst = jnp.concatenate([dst, jnp.zeros((pad,), jnp.int32)])
    w = jnp.concatenate([w, jnp.zeros((pad,), jnp.float32)])
    src = src.reshape(NS, NCH * CH)
    dst = dst.reshape(NS, NCH, CH)
    w = w.reshape(NS, NCH * CH)

    nf_halves = jnp.zeros((NC, N_PAD, DH), jnp.float32).at[:, :N_NODES].set(
        n_feat.reshape(N_NODES, NC, DH).swapaxes(0, 1))

    partials = _sc_aggregate(nf_halves, src, dst, w)[:, :N_NODES, :]

    m = W_in.T + jnp.eye(D_FEAT, dtype=jnp.float32)
    out = pl.pallas_call(
        _tc_body,
        out_shape=jax.ShapeDtypeStruct((N_NODES, D_FEAT), jnp.float32),
    )(partials, m, b_in.reshape(1, D_FEAT))
    return out
